# Initial kernel scaffold; baseline (speedup 1.0000x reference)
import jax
import jax.numpy as jnp
from jax.experimental import pallas as pl


def kernel(x_stroke, sketch_strokes_id, ei_intersects, ei_temp_previous, ei_represented_by, ei_brepcoplanar, ei_strokecoplanar, W_nbr_intersects, W_self_intersects, b_intersects, W_nbr_temp_previous, W_self_temp_previous, b_temp_previous, W_nbr_represented_by, W_self_represented_by, b_represented_by, W_nbr_brepcoplanar, W_self_brepcoplanar, b_brepcoplanar, W_nbr_strokecoplanar, W_self_strokecoplanar, b_strokecoplanar, W_local, b_local, W_dec1, b_dec1, W_dec2, b_dec2):
    raise NotImplementedError("write your pallas kernel here")


# speedup vs baseline: 1.0396x; 1.0396x over previous
import functools

import jax
import jax.numpy as jnp
from jax.experimental import pallas as pl

N = 10000
D = 32
E = 320000

_REL_ORDER = ["intersects", "temp_previous", "represented_by", "brepcoplanar", "strokecoplanar"]
_REL_AGG = {"intersects": "mean", "temp_previous": "add", "represented_by": "mean",
            "brepcoplanar": "max", "strokecoplanar": "max"}


def _prologue_body(xs_ref, id_ref, wn0, wn1, wn2, wn3, wn4, wself_ref, bsum_ref,
                   x_out, y0, y1, y2, y3, y4, self_out):
    x = xs_ref[...] + xs_ref[...] * id_ref[...]
    x_out[...] = x
    y0[...] = jnp.dot(x, wn0[...], preferred_element_type=jnp.float32)
    y1[...] = jnp.dot(x, wn1[...], preferred_element_type=jnp.float32)
    y2[...] = jnp.dot(x, wn2[...], preferred_element_type=jnp.float32)
    y3[...] = jnp.dot(x, wn3[...], preferred_element_type=jnp.float32)
    y4[...] = jnp.dot(x, wn4[...], preferred_element_type=jnp.float32)
    self_out[...] = jnp.dot(x, wself_ref[...], preferred_element_type=jnp.float32) + bsum_ref[...]


def _epilogue_body(x_ref, self_ref, s_int, c_int, s_tmp, s_rep, c_rep, m_brep, m_strk,
                   wl_ref, bl_ref, w1_ref, b1_ref, w2_ref, b2_ref, o_ref):
    out = self_ref[...]
    out += s_int[...] / jnp.clip(c_int[...], 1.0, None)
    out += s_tmp[...]
    out += s_rep[...] / jnp.clip(c_rep[...], 1.0, None)
    mb = m_brep[...]
    out += jnp.where(jnp.isfinite(mb), mb, 0.0)
    ms = m_strk[...]
    out += jnp.where(jnp.isfinite(ms), ms, 0.0)
    h = x_ref[...] + jax.nn.relu(out)
    feat = jnp.dot(h, wl_ref[...], preferred_element_type=jnp.float32) + bl_ref[...]
    dvec = jax.nn.relu(jnp.dot(feat, w1_ref[...], preferred_element_type=jnp.float32) + b1_ref[...])
    logit = jnp.dot(dvec, w2_ref[...], preferred_element_type=jnp.float32) + b2_ref[...]
    o_ref[...] = jax.nn.sigmoid(logit)


def kernel(x_stroke, sketch_strokes_id, ei_intersects, ei_temp_previous, ei_represented_by, ei_brepcoplanar, ei_strokecoplanar, W_nbr_intersects, W_self_intersects, b_intersects, W_nbr_temp_previous, W_self_temp_previous, b_temp_previous, W_nbr_represented_by, W_self_represented_by, b_represented_by, W_nbr_brepcoplanar, W_self_brepcoplanar, b_brepcoplanar, W_nbr_strokecoplanar, W_self_strokecoplanar, b_strokecoplanar, W_local, b_local, W_dec1, b_dec1, W_dec2, b_dec2):
    f32 = jnp.float32
    w_self_sum = (W_self_intersects + W_self_temp_previous + W_self_represented_by
                  + W_self_brepcoplanar + W_self_strokecoplanar)
    b_sum = (b_intersects + b_temp_previous + b_represented_by
             + b_brepcoplanar + b_strokecoplanar).reshape(1, D)

    shp = jax.ShapeDtypeStruct((N, D), f32)
    x, y_int, y_tmp, y_rep, y_brep, y_strk, selfterm = pl.pallas_call(
        _prologue_body,
        out_shape=[shp] * 7,
    )(x_stroke, sketch_strokes_id, W_nbr_intersects, W_nbr_temp_previous,
      W_nbr_represented_by, W_nbr_brepcoplanar, W_nbr_strokecoplanar,
      w_self_sum, b_sum)

    ones = jnp.ones((E, 1), f32)
    s_int = jax.ops.segment_sum(y_int[ei_intersects[0]], ei_intersects[1], num_segments=N)
    c_int = jax.ops.segment_sum(ones, ei_intersects[1], num_segments=N)
    s_tmp = jax.ops.segment_sum(y_tmp[ei_temp_previous[0]], ei_temp_previous[1], num_segments=N)
    s_rep = jax.ops.segment_sum(y_rep[ei_represented_by[0]], ei_represented_by[1], num_segments=N)
    c_rep = jax.ops.segment_sum(ones, ei_represented_by[1], num_segments=N)
    m_brep = jax.ops.segment_max(y_brep[ei_brepcoplanar[0]], ei_brepcoplanar[1], num_segments=N)
    m_strk = jax.ops.segment_max(y_strk[ei_strokecoplanar[0]], ei_strokecoplanar[1], num_segments=N)

    out = pl.pallas_call(
        _epilogue_body,
        out_shape=jax.ShapeDtypeStruct((N, 1), f32),
    )(x, selfterm, s_int, c_int, s_tmp, s_rep, c_rep, m_brep, m_strk,
      W_local, b_local.reshape(1, 64), W_dec1, b_dec1.reshape(1, 64),
      W_dec2, b_dec2.reshape(1, 1))
    return out


# speedup vs baseline: 2.3022x; 2.2146x over previous
import jax
import jax.numpy as jnp
from jax import lax
from jax.experimental import pallas as pl
from jax.experimental.pallas import tpu as pltpu
from jax.experimental.pallas import tpu_sc as plsc

N = 10000
D = 32
E = 320000

NC = 2
NS = 16
NW = NC * NS
WIN = 1024
NWIN = 10
EPW = WIN * NWIN
E_PAD = EPW * NW
N_PAD = 10112
ROWS_PER_SUB = N_PAD // NS


def _prologue_body(xs_ref, id_ref, wn0, wn1, wn2, wn3, wn4,
                   ws0, ws1, ws2, ws3, ws4, bsum_ref,
                   x_out, y0, y1, y2, y3, y4, self_out):
    x = xs_ref[...] + xs_ref[...] * id_ref[...]
    x_out[...] = x
    y0[...] = jnp.dot(x, wn0[...], preferred_element_type=jnp.float32)
    y1[...] = jnp.dot(x, wn1[...], preferred_element_type=jnp.float32)
    y2[...] = jnp.dot(x, wn2[...], preferred_element_type=jnp.float32)
    y3[...] = jnp.dot(x, wn3[...], preferred_element_type=jnp.float32)
    y4[...] = jnp.dot(x, wn4[...], preferred_element_type=jnp.float32)
    self_out[...] = (jnp.dot(x, ws0[...], preferred_element_type=jnp.float32)
                     + jnp.dot(x, ws1[...], preferred_element_type=jnp.float32)
                     + jnp.dot(x, ws2[...], preferred_element_type=jnp.float32)
                     + jnp.dot(x, ws3[...], preferred_element_type=jnp.float32)
                     + jnp.dot(x, ws4[...], preferred_element_type=jnp.float32)
                     + bsum_ref[...])


NSUB = WIN // 128


def _sc_scatter_body(y_int, y_tmp, y_rep, ei_int, ei_tmp, ei_rep,
                     zeros32, zeros16, ones_h,
                     out_int, out_tmp, out_rep, out_cint, out_crep,
                     srcv, dstv, rows, onesv, gsem, ssem,
                     agg_int, agg_tmp, agg_rep, cnt_int, cnt_rep):
    c = lax.axis_index("c")
    s = lax.axis_index("s")
    wid = s * NC + c

    @pl.when(s == 0)
    def _():
        pltpu.sync_copy(zeros32, agg_int)
        pltpu.sync_copy(zeros32, agg_tmp)
        pltpu.sync_copy(zeros32, agg_rep)
        pltpu.sync_copy(zeros16, cnt_int)
        pltpu.sync_copy(zeros16, cnt_rep)

    pltpu.sync_copy(ones_h, onesv)
    plsc.subcore_barrier()

    row_base = wid * (EPW // 128)
    for y, ei, agg, cnt in ((y_int, ei_int, agg_int, cnt_int),
                            (y_tmp, ei_tmp, agg_tmp, None),
                            (y_rep, ei_rep, agg_rep, cnt_rep)):
        for i in range(NWIN):
            r0 = row_base + i * NSUB
            pltpu.sync_copy(ei.at[0, pl.ds(r0, NSUB), :], srcv)
            pltpu.sync_copy(ei.at[1, pl.ds(r0, NSUB), :], dstv)
            gathers = [
                pltpu.async_copy(y.at[srcv.at[j]], rows.at[pl.ds(j * 128, 128)], gsem)
                for j in range(NSUB)
            ]
            for g in gathers:
                g.wait()
            scatters = []
            for j in range(NSUB):
                scatters.append(pltpu.async_copy(
                    rows.at[pl.ds(j * 128, 128)], agg.at[dstv.at[j]], ssem, add=True))
                if cnt is not None:
                    scatters.append(pltpu.async_copy(
                        onesv, cnt.at[dstv.at[j]], ssem, add=True))
            for sc_ in scatters:
                sc_.wait()

    plsc.subcore_barrier()
    r0 = s * ROWS_PER_SUB
    pltpu.sync_copy(agg_int.at[pl.ds(r0, ROWS_PER_SUB)], out_int.at[c, pl.ds(r0, ROWS_PER_SUB)])
    pltpu.sync_copy(agg_tmp.at[pl.ds(r0, ROWS_PER_SUB)], out_tmp.at[c, pl.ds(r0, ROWS_PER_SUB)])
    pltpu.sync_copy(agg_rep.at[pl.ds(r0, ROWS_PER_SUB)], out_rep.at[c, pl.ds(r0, ROWS_PER_SUB)])
    pltpu.sync_copy(cnt_int.at[pl.ds(r0, ROWS_PER_SUB)], out_cint.at[c, pl.ds(r0, ROWS_PER_SUB)])
    pltpu.sync_copy(cnt_rep.at[pl.ds(r0, ROWS_PER_SUB)], out_crep.at[c, pl.ds(r0, ROWS_PER_SUB)])


def _epilogue_body(x_ref, self_ref, s_int, c_int, s_tmp, s_rep, c_rep, m_brep, m_strk,
                   wl_ref, bl_ref, w1_ref, b1_ref, w2_ref, b2_ref, o_ref):
    out = self_ref[...]
    out += (s_int[0] + s_int[1]) / jnp.clip(c_int[...], 1.0, None)
    out += s_tmp[0] + s_tmp[1]
    out += (s_rep[0] + s_rep[1]) / jnp.clip(c_rep[...], 1.0, None)
    mb = m_brep[...]
    out += jnp.where(jnp.isfinite(mb), mb, 0.0)
    ms = m_strk[...]
    out += jnp.where(jnp.isfinite(ms), ms, 0.0)
    h = x_ref[...] + jax.nn.relu(out)
    feat = jnp.dot(h, wl_ref[...], preferred_element_type=jnp.float32) + bl_ref[...]
    dvec = jax.nn.relu(jnp.dot(feat, w1_ref[...], preferred_element_type=jnp.float32) + b1_ref[...])
    logit = jnp.dot(dvec, w2_ref[...], preferred_element_type=jnp.float32) + b2_ref[...]
    o_ref[...] = jax.nn.sigmoid(logit)


def _pad_edges(ei):
    pad = E_PAD - E
    fill = jnp.concatenate([jnp.zeros((1, pad), jnp.int32),
                            jnp.full((1, pad), N, jnp.int32)], axis=0)
    return jnp.concatenate([ei, fill], axis=1).reshape(2, E_PAD // 128, 128)


def kernel(x_stroke, sketch_strokes_id, ei_intersects, ei_temp_previous, ei_represented_by, ei_brepcoplanar, ei_strokecoplanar, W_nbr_intersects, W_self_intersects, b_intersects, W_nbr_temp_previous, W_self_temp_previous, b_temp_previous, W_nbr_represented_by, W_self_represented_by, b_represented_by, W_nbr_brepcoplanar, W_self_brepcoplanar, b_brepcoplanar, W_nbr_strokecoplanar, W_self_strokecoplanar, b_strokecoplanar, W_local, b_local, W_dec1, b_dec1, W_dec2, b_dec2):
    f32 = jnp.float32
    b_sum = (b_intersects + b_temp_previous + b_represented_by
             + b_brepcoplanar + b_strokecoplanar).reshape(1, D)

    shp = jax.ShapeDtypeStruct((N, D), f32)
    PBLK = 2000
    prow = pl.BlockSpec((PBLK, D), lambda i: (i, 0))
    x, y_int, y_tmp, y_rep, y_brep, y_strk, selfterm = pl.pallas_call(
        _prologue_body,
        grid=(N // PBLK,),
        in_specs=[
            prow,
            pl.BlockSpec((PBLK, 1), lambda i: (i, 0)),
        ] + [pl.BlockSpec((D, D), lambda i: (0, 0))] * 10
          + [pl.BlockSpec((1, D), lambda i: (0, 0))],
        out_specs=[prow] * 7,
        out_shape=[shp] * 7,
    )(x_stroke, sketch_strokes_id, W_nbr_intersects, W_nbr_temp_previous,
      W_nbr_represented_by, W_nbr_brepcoplanar, W_nbr_strokecoplanar,
      W_self_intersects, W_self_temp_previous, W_self_represented_by,
      W_self_brepcoplanar, W_self_strokecoplanar, b_sum)

    def pad_tbl(y):
        return jnp.concatenate([y, jnp.zeros((N_PAD - N, D), f32)], axis=0)

    y_int_p, y_tmp_p, y_rep_p = pad_tbl(y_int), pad_tbl(y_tmp), pad_tbl(y_rep)
    ei_int_p = _pad_edges(ei_intersects)
    ei_tmp_p = _pad_edges(ei_temp_previous)
    ei_rep_p = _pad_edges(ei_represented_by)

    mesh = plsc.VectorSubcoreMesh(core_axis_name="c", subcore_axis_name="s")
    sc_fn = pl.kernel(
        _sc_scatter_body,
        mesh=mesh,
        compiler_params=pltpu.CompilerParams(use_tc_tiling_on_sc=False),
        out_type=[
            jax.ShapeDtypeStruct((NC, N_PAD, D), f32),
            jax.ShapeDtypeStruct((NC, N_PAD, D), f32),
            jax.ShapeDtypeStruct((NC, N_PAD, D), f32),
            jax.ShapeDtypeStruct((NC, N_PAD, 16), f32),
            jax.ShapeDtypeStruct((NC, N_PAD, 16), f32),
        ],
        scratch_types=[
            pltpu.VMEM((NSUB, 128), jnp.int32),
            pltpu.VMEM((NSUB, 128), jnp.int32),
            pltpu.VMEM((WIN, D), f32),
            pltpu.VMEM((128, 16), f32),
            pltpu.SemaphoreType.DMA,
            pltpu.SemaphoreType.DMA,
            pltpu.VMEM_SHARED((N_PAD, D), f32),
            pltpu.VMEM_SHARED((N_PAD, D), f32),
            pltpu.VMEM_SHARED((N_PAD, D), f32),
            pltpu.VMEM_SHARED((N_PAD, 16), f32),
            pltpu.VMEM_SHARED((N_PAD, 16), f32),
        ],
    )
    zeros32 = jnp.zeros((N_PAD, D), f32)
    zeros16 = jnp.zeros((N_PAD, 16), f32)
    ones_h = jnp.ones((128, 16), f32)
    s_int, s_tmp, s_rep, c_int, c_rep = sc_fn(
        y_int_p, y_tmp_p, y_rep_p, ei_int_p, ei_tmp_p, ei_rep_p,
        zeros32, zeros16, ones_h)

    m_brep = jax.ops.segment_max(y_brep[ei_brepcoplanar[0]], ei_brepcoplanar[1], num_segments=N)
    m_strk = jax.ops.segment_max(y_strk[ei_strokecoplanar[0]], ei_strokecoplanar[1], num_segments=N)

    c_int_s = (c_int[0] + c_int[1])[:N, :1]
    c_rep_s = (c_rep[0] + c_rep[1])[:N, :1]

    BLK = 2000
    grid = N // BLK
    row_blk = pl.BlockSpec((BLK, D), lambda i: (i, 0))
    par_blk = pl.BlockSpec((NC, BLK, D), lambda i: (0, i, 0))
    cnt_blk = pl.BlockSpec((BLK, 1), lambda i: (i, 0))
    out = pl.pallas_call(
        _epilogue_body,
        grid=(grid,),
        in_specs=[
            row_blk, row_blk,
            par_blk, cnt_blk, par_blk, par_blk, cnt_blk,
            row_blk, row_blk,
            pl.BlockSpec((D, 64), lambda i: (0, 0)),
            pl.BlockSpec((1, 64), lambda i: (0, 0)),
            pl.BlockSpec((64, 64), lambda i: (0, 0)),
            pl.BlockSpec((1, 64), lambda i: (0, 0)),
            pl.BlockSpec((64, 1), lambda i: (0, 0)),
            pl.BlockSpec((1, 1), lambda i: (0, 0)),
        ],
        out_specs=pl.BlockSpec((BLK, 1), lambda i: (i, 0)),
        out_shape=jax.ShapeDtypeStruct((N, 1), f32),
    )(x, selfterm, s_int, c_int_s, s_tmp, s_rep, c_rep_s, m_brep, m_strk,
      W_local, b_local.reshape(1, 64), W_dec1, b_dec1.reshape(1, 64),
      W_dec2, b_dec2.reshape(1, 1))
    return out


# speedup vs baseline: 3.1670x; 1.3756x over previous
import jax
import jax.numpy as jnp
from jax import lax
from jax.experimental import pallas as pl
from jax.experimental.pallas import tpu as pltpu
from jax.experimental.pallas import tpu_sc as plsc

N = 10000
D = 32
E = 320000

NC = 2
NS = 16
NW = NC * NS
WIN = 1024
NWIN = 10
EPW = WIN * NWIN
E_PAD = EPW * NW
N_PAD = 10112
ROWS_PER_SUB = N_PAD // NS


def _prologue_body(xs_ref, id_ref, wn0, wn1, wn2, wn3, wn4,
                   ws0, ws1, ws2, ws3, ws4, bsum_ref,
                   x_out, y0, y1, y2, y3, y4, self_out):
    x = xs_ref[...] + xs_ref[...] * id_ref[...]
    x_out[...] = x
    y0[...] = jnp.dot(x, wn0[...], preferred_element_type=jnp.float32)
    y1[...] = jnp.dot(x, wn1[...], preferred_element_type=jnp.float32)
    y2[...] = jnp.dot(x, wn2[...], preferred_element_type=jnp.float32)
    y3[...] = jnp.dot(x, wn3[...], preferred_element_type=jnp.float32)
    y4[...] = jnp.dot(x, wn4[...], preferred_element_type=jnp.float32)
    self_out[...] = (jnp.dot(x, ws0[...], preferred_element_type=jnp.float32)
                     + jnp.dot(x, ws1[...], preferred_element_type=jnp.float32)
                     + jnp.dot(x, ws2[...], preferred_element_type=jnp.float32)
                     + jnp.dot(x, ws3[...], preferred_element_type=jnp.float32)
                     + jnp.dot(x, ws4[...], preferred_element_type=jnp.float32)
                     + bsum_ref[...])


NSUB = WIN // 128


def _sc_scatter_body(y_int, y_tmp, y_rep, ei_int, ei_tmp, ei_rep,
                     zeros32, zeros16, ones_h,
                     out_int, out_tmp, out_rep, out_cint, out_crep,
                     srcv, dstv, rows, onesv, gsem, ssem,
                     agg_int, agg_tmp, agg_rep, cnt_int, cnt_rep):
    c = lax.axis_index("c")
    s = lax.axis_index("s")
    wid = s * NC + c

    @pl.when(s == 0)
    def _():
        pltpu.sync_copy(zeros32, agg_int)
        pltpu.sync_copy(zeros32, agg_tmp)
        pltpu.sync_copy(zeros32, agg_rep)
        pltpu.sync_copy(zeros16, cnt_int)
        pltpu.sync_copy(zeros16, cnt_rep)

    pltpu.sync_copy(ones_h, onesv)
    plsc.subcore_barrier()

    row_base = wid * (EPW // 128)
    for y, ei, agg, cnt in ((y_int, ei_int, agg_int, cnt_int),
                            (y_tmp, ei_tmp, agg_tmp, None),
                            (y_rep, ei_rep, agg_rep, cnt_rep)):
        for i in range(NWIN):
            r0 = row_base + i * NSUB
            pltpu.sync_copy(ei.at[0, pl.ds(r0, NSUB), :], srcv)
            pltpu.sync_copy(ei.at[1, pl.ds(r0, NSUB), :], dstv)
            gathers = [
                pltpu.async_copy(y.at[srcv.at[j]], rows.at[pl.ds(j * 128, 128)], gsem)
                for j in range(NSUB)
            ]
            for g in gathers:
                g.wait()
            scatters = []
            for j in range(NSUB):
                scatters.append(pltpu.async_copy(
                    rows.at[pl.ds(j * 128, 128)], agg.at[dstv.at[j]], ssem, add=True))
                if cnt is not None:
                    scatters.append(pltpu.async_copy(
                        onesv, cnt.at[dstv.at[j]], ssem, add=True))
            for sc_ in scatters:
                sc_.wait()

    plsc.subcore_barrier()
    r0 = s * ROWS_PER_SUB
    pltpu.sync_copy(agg_int.at[pl.ds(r0, ROWS_PER_SUB)], out_int.at[c, pl.ds(r0, ROWS_PER_SUB)])
    pltpu.sync_copy(agg_tmp.at[pl.ds(r0, ROWS_PER_SUB)], out_tmp.at[c, pl.ds(r0, ROWS_PER_SUB)])
    pltpu.sync_copy(agg_rep.at[pl.ds(r0, ROWS_PER_SUB)], out_rep.at[c, pl.ds(r0, ROWS_PER_SUB)])
    pltpu.sync_copy(cnt_int.at[pl.ds(r0, ROWS_PER_SUB)], out_cint.at[c, pl.ds(r0, ROWS_PER_SUB)])
    pltpu.sync_copy(cnt_rep.at[pl.ds(r0, ROWS_PER_SUB)], out_crep.at[c, pl.ds(r0, ROWS_PER_SUB)])


NPB = N_PAD // NS
CAP_B = 1024
EPW_MX = (E // NC) // NS
MXWIN = 2000
MAGIC = 6637


def _sc_max_body(y_brep, y_strk, ei_brep, ei_strk, neginf, initsrc,
                 out_brep, out_strk,
                 srcw, dstw, src_store, dst_store, cnt16, cnts256,
                 blk_src, blk_dst, rows, table, gsem,
                 exch_src, exch_dst, cnts_sp):
    c = lax.axis_index("c")
    s = lax.axis_index("s")
    i32 = jnp.int32
    iota = lax.iota(i32, 16)
    d0, _ = plsc.scan_count(iota)
    dupbase = jnp.min(d0)
    cvecs = [lax.full((16,), cc, i32) for cc in range(D)]

    for y, ei, out in ((y_brep, ei_brep, out_brep), (y_strk, ei_strk, out_strk)):
        pltpu.sync_copy(neginf, table)
        pltpu.sync_copy(initsrc, src_store)
        pltpu.sync_copy(initsrc, dst_store)
        cnt16[...] = jnp.zeros((16,), i32)

        for win in range(EPW_MX // MXWIN):
            eb = c * (E // NC) + s * EPW_MX + win * MXWIN
            pltpu.sync_copy(ei.at[0, pl.ds(eb, MXWIN)], srcw)
            pltpu.sync_copy(ei.at[1, pl.ds(eb, MXWIN)], dstw)

            def chunk(k, _):
                off = k * 16
                dstv = dstw[pl.ds(off, 16)]
                srcv = srcw[pl.ds(off, 16)]
                owner = (dstv * MAGIC) >> 22
                dup, last = plsc.scan_count(owner)
                dupz = dup - dupbase
                cur = plsc.load_gather(cnt16, [owner])
                slot = jnp.minimum(cur + dupz, CAP_B - 1)
                addr = owner * CAP_B + slot
                plsc.store_scatter(src_store, [addr], srcv)
                plsc.store_scatter(dst_store, [addr], dstv)
                plsc.store_scatter(cnt16, [owner],
                                   jnp.minimum(cur + dupz + 1, CAP_B), mask=last)
                return 0

            lax.fori_loop(0, MXWIN // 16, chunk, 0)

        pltpu.sync_copy(src_store, exch_src.at[pl.ds(s * (16 * CAP_B), 16 * CAP_B)])
        pltpu.sync_copy(dst_store, exch_dst.at[pl.ds(s * (16 * CAP_B), 16 * CAP_B)])
        pltpu.sync_copy(cnt16, cnts_sp.at[pl.ds(s * 16, 16)])
        plsc.subcore_barrier()

        pltpu.sync_copy(cnts_sp, cnts256)
        mycnts = plsc.load_gather(cnts256, [iota * 16 + s])

        def vbody(v, _):
            cnt_v = jnp.max(jnp.where(iota == v, mycnts, 0))
            base = (v * 16 + s) * CAP_B
            pltpu.sync_copy(exch_src.at[pl.ds(base, CAP_B)], blk_src)
            pltpu.sync_copy(exch_dst.at[pl.ds(base, CAP_B)], blk_dst)
            gathers = [
                pltpu.async_copy(y.at[blk_src.at[pl.ds(j * 128, 128)]],
                                 rows.at[pl.ds(j * 128, 128)], gsem)
                for j in range(CAP_B // 128)
            ]
            for g in gathers:
                g.wait()

            def rmw(k, _):
                off = k * 16
                valid = iota < (cnt_v - off)
                dstv = blk_dst[pl.ds(off, 16)]
                local = dstv - s * NPB
                dup, _l = plsc.scan_count(local, mask=valid)
                dupz = dup - dupbase
                nr = jnp.max(jnp.where(valid, dupz, 0)) + 1

                def rnd(r, _):
                    mr = valid & (dupz == r)
                    for cc in range(D):
                        colv = plsc.load_gather(rows, [off + iota, cvecs[cc]])
                        tv = plsc.load_gather(table, [local, cvecs[cc]], mask=mr)
                        nv = jnp.maximum(tv, colv)
                        plsc.store_scatter(table, [local, cvecs[cc]], nv, mask=mr)
                    return 0

                lax.fori_loop(0, nr, rnd, 0)
                return 0

            lax.fori_loop(0, (cnt_v + 15) // 16, rmw, 0)
            return 0

        lax.fori_loop(0, NS, vbody, 0)

        pltpu.sync_copy(table, out.at[c, pl.ds(s * NPB, NPB)])
        plsc.subcore_barrier()


def _epilogue_body(x_ref, self_ref, s_int, c_int, s_tmp, s_rep, c_rep, m_brep, m_strk,
                   wl_ref, bl_ref, w1_ref, b1_ref, w2_ref, b2_ref, o_ref):
    out = self_ref[...]
    out += (s_int[0] + s_int[1]) / jnp.clip(c_int[...], 1.0, None)
    out += s_tmp[0] + s_tmp[1]
    out += (s_rep[0] + s_rep[1]) / jnp.clip(c_rep[...], 1.0, None)
    mb = jnp.maximum(m_brep[0], m_brep[1])
    out += jnp.where(jnp.isfinite(mb), mb, 0.0)
    ms = jnp.maximum(m_strk[0], m_strk[1])
    out += jnp.where(jnp.isfinite(ms), ms, 0.0)
    h = x_ref[...] + jax.nn.relu(out)
    feat = jnp.dot(h, wl_ref[...], preferred_element_type=jnp.float32) + bl_ref[...]
    dvec = jax.nn.relu(jnp.dot(feat, w1_ref[...], preferred_element_type=jnp.float32) + b1_ref[...])
    logit = jnp.dot(dvec, w2_ref[...], preferred_element_type=jnp.float32) + b2_ref[...]
    o_ref[...] = jax.nn.sigmoid(logit)


def _pad_edges(ei):
    pad = E_PAD - E
    fill = jnp.concatenate([jnp.zeros((1, pad), jnp.int32),
                            jnp.full((1, pad), N, jnp.int32)], axis=0)
    return jnp.concatenate([ei, fill], axis=1).reshape(2, E_PAD // 128, 128)


def kernel(x_stroke, sketch_strokes_id, ei_intersects, ei_temp_previous, ei_represented_by, ei_brepcoplanar, ei_strokecoplanar, W_nbr_intersects, W_self_intersects, b_intersects, W_nbr_temp_previous, W_self_temp_previous, b_temp_previous, W_nbr_represented_by, W_self_represented_by, b_represented_by, W_nbr_brepcoplanar, W_self_brepcoplanar, b_brepcoplanar, W_nbr_strokecoplanar, W_self_strokecoplanar, b_strokecoplanar, W_local, b_local, W_dec1, b_dec1, W_dec2, b_dec2):
    f32 = jnp.float32
    b_sum = (b_intersects + b_temp_previous + b_represented_by
             + b_brepcoplanar + b_strokecoplanar).reshape(1, D)

    shp = jax.ShapeDtypeStruct((N, D), f32)
    PBLK = 2000
    prow = pl.BlockSpec((PBLK, D), lambda i: (i, 0))
    x, y_int, y_tmp, y_rep, y_brep, y_strk, selfterm = pl.pallas_call(
        _prologue_body,
        grid=(N // PBLK,),
        in_specs=[
            prow,
            pl.BlockSpec((PBLK, 1), lambda i: (i, 0)),
        ] + [pl.BlockSpec((D, D), lambda i: (0, 0))] * 10
          + [pl.BlockSpec((1, D), lambda i: (0, 0))],
        out_specs=[prow] * 7,
        out_shape=[shp] * 7,
    )(x_stroke, sketch_strokes_id, W_nbr_intersects, W_nbr_temp_previous,
      W_nbr_represented_by, W_nbr_brepcoplanar, W_nbr_strokecoplanar,
      W_self_intersects, W_self_temp_previous, W_self_represented_by,
      W_self_brepcoplanar, W_self_strokecoplanar, b_sum)

    def pad_tbl(y):
        return jnp.concatenate([y, jnp.zeros((N_PAD - N, D), f32)], axis=0)

    y_int_p, y_tmp_p, y_rep_p = pad_tbl(y_int), pad_tbl(y_tmp), pad_tbl(y_rep)
    ei_int_p = _pad_edges(ei_intersects)
    ei_tmp_p = _pad_edges(ei_temp_previous)
    ei_rep_p = _pad_edges(ei_represented_by)

    mesh = plsc.VectorSubcoreMesh(core_axis_name="c", subcore_axis_name="s")
    sc_fn = pl.kernel(
        _sc_scatter_body,
        mesh=mesh,
        compiler_params=pltpu.CompilerParams(use_tc_tiling_on_sc=False),
        out_type=[
            jax.ShapeDtypeStruct((NC, N_PAD, D), f32),
            jax.ShapeDtypeStruct((NC, N_PAD, D), f32),
            jax.ShapeDtypeStruct((NC, N_PAD, D), f32),
            jax.ShapeDtypeStruct((NC, N_PAD, 16), f32),
            jax.ShapeDtypeStruct((NC, N_PAD, 16), f32),
        ],
        scratch_types=[
            pltpu.VMEM((NSUB, 128), jnp.int32),
            pltpu.VMEM((NSUB, 128), jnp.int32),
            pltpu.VMEM((WIN, D), f32),
            pltpu.VMEM((128, 16), f32),
            pltpu.SemaphoreType.DMA,
            pltpu.SemaphoreType.DMA,
            pltpu.VMEM_SHARED((N_PAD, D), f32),
            pltpu.VMEM_SHARED((N_PAD, D), f32),
            pltpu.VMEM_SHARED((N_PAD, D), f32),
            pltpu.VMEM_SHARED((N_PAD, 16), f32),
            pltpu.VMEM_SHARED((N_PAD, 16), f32),
        ],
    )
    zeros32 = jnp.zeros((N_PAD, D), f32)
    zeros16 = jnp.zeros((N_PAD, 16), f32)
    ones_h = jnp.ones((128, 16), f32)
    s_int, s_tmp, s_rep, c_int, c_rep = sc_fn(
        y_int_p, y_tmp_p, y_rep_p, ei_int_p, ei_tmp_p, ei_rep_p,
        zeros32, zeros16, ones_h)

    max_fn = pl.kernel(
        _sc_max_body,
        mesh=mesh,
        compiler_params=pltpu.CompilerParams(use_tc_tiling_on_sc=False,
                                             needs_layout_passes=False),
        out_type=[
            jax.ShapeDtypeStruct((NC, N_PAD, D), f32),
            jax.ShapeDtypeStruct((NC, N_PAD, D), f32),
        ],
        scratch_types=[
            pltpu.VMEM((MXWIN,), jnp.int32),
            pltpu.VMEM((MXWIN,), jnp.int32),
            pltpu.VMEM((16 * CAP_B,), jnp.int32),
            pltpu.VMEM((16 * CAP_B,), jnp.int32),
            pltpu.VMEM((16,), jnp.int32),
            pltpu.VMEM((256,), jnp.int32),
            pltpu.VMEM((CAP_B,), jnp.int32),
            pltpu.VMEM((CAP_B,), jnp.int32),
            pltpu.VMEM((CAP_B, D), f32),
            pltpu.VMEM((NPB, D), f32),
            pltpu.SemaphoreType.DMA,
            pltpu.VMEM_SHARED((NS * 16 * CAP_B,), jnp.int32),
            pltpu.VMEM_SHARED((NS * 16 * CAP_B,), jnp.int32),
            pltpu.VMEM_SHARED((256,), jnp.int32),
        ],
    )
    neginf = jnp.full((NPB, D), -jnp.inf, f32)
    initsrc = (jnp.arange(16 * CAP_B, dtype=jnp.int32) * 37) % N
    m_brep, m_strk = max_fn(y_brep, y_strk, ei_brepcoplanar, ei_strokecoplanar,
                            neginf, initsrc)

    c_int_s = (c_int[0] + c_int[1])[:N, :1]
    c_rep_s = (c_rep[0] + c_rep[1])[:N, :1]

    BLK = 2000
    grid = N // BLK
    row_blk = pl.BlockSpec((BLK, D), lambda i: (i, 0))
    par_blk = pl.BlockSpec((NC, BLK, D), lambda i: (0, i, 0))
    cnt_blk = pl.BlockSpec((BLK, 1), lambda i: (i, 0))
    out = pl.pallas_call(
        _epilogue_body,
        grid=(grid,),
        in_specs=[
            row_blk, row_blk,
            par_blk, cnt_blk, par_blk, par_blk, cnt_blk,
            par_blk, par_blk,
            pl.BlockSpec((D, 64), lambda i: (0, 0)),
            pl.BlockSpec((1, 64), lambda i: (0, 0)),
            pl.BlockSpec((64, 64), lambda i: (0, 0)),
            pl.BlockSpec((1, 64), lambda i: (0, 0)),
            pl.BlockSpec((64, 1), lambda i: (0, 0)),
            pl.BlockSpec((1, 1), lambda i: (0, 0)),
        ],
        out_specs=pl.BlockSpec((BLK, 1), lambda i: (i, 0)),
        out_shape=jax.ShapeDtypeStruct((N, 1), f32),
    )(x, selfterm, s_int, c_int_s, s_tmp, s_rep, c_rep_s, m_brep, m_strk,
      W_local, b_local.reshape(1, 64), W_dec1, b_dec1.reshape(1, 64),
      W_dec2, b_dec2.reshape(1, 1))
    return out
